# Initial kernel scaffold; baseline (speedup 1.0000x reference)
import jax
import jax.numpy as jnp
from jax.experimental import pallas as pl


def kernel(x, edge_index, batch, num_graphs, W1_0, b1_0, W2_0, b2_0, g_0, be_0, W1_1, b1_1, W2_1, b2_1, g_1, be_1, W1_2, b1_2, W2_2, b2_2, g_2, be_2, P1, pb1, P2, pb2):
    raise NotImplementedError("write your pallas kernel here")


# speedup vs baseline: 3.9779x; 3.9779x over previous
import functools

import jax
import jax.numpy as jnp
from jax import lax
from jax.experimental import pallas as pl
from jax.experimental.pallas import tpu as pltpu
from jax.experimental.pallas import tpu_sc as plsc

_N = 10000
_E = 320000
_D = 128
_G = 128
_NC = 2
_NS = 16
_NW = _NC * _NS
_EPW = _E // _NW
_K = 80
_NCHUNK = _EPW // _K
_RPS = 624
_TAIL = _N - _NS * _RPS

_RB = 400
_NRB = _N // _RB


def _segsum_body(h_hbm, src_hbm, dst_hbm, zeros_hbm, out_hbm,
                 acc_sh, sidx_v, didx_v, rows_v, sem):
    c = lax.axis_index("c")
    s = lax.axis_index("s")
    pltpu.sync_copy(zeros_hbm.at[pl.ds(s * _RPS, _RPS)],
                    acc_sh.at[pl.ds(s * _RPS, _RPS)])

    @pl.when(s == _NS - 1)
    def _():
        pltpu.sync_copy(zeros_hbm.at[pl.ds(_NS * _RPS, _TAIL)],
                        acc_sh.at[pl.ds(_NS * _RPS, _TAIL)])

    plsc.subcore_barrier()

    wid = s * _NC + c
    base = wid * _EPW

    def chunk(j, carry):
        off = pl.multiple_of(base + j * _K, 8)
        pltpu.sync_copy(src_hbm.at[pl.ds(off, _K)], sidx_v)
        pltpu.sync_copy(dst_hbm.at[pl.ds(off, _K)], didx_v)
        pltpu.async_copy(h_hbm.at[sidx_v], rows_v, sem).wait()
        pltpu.sync_copy(rows_v, acc_sh.at[didx_v], add=True)
        return carry

    lax.fori_loop(0, _NCHUNK, chunk, 0)
    plsc.subcore_barrier()
    pltpu.sync_copy(acc_sh.at[pl.ds(s * _RPS, _RPS)],
                    out_hbm.at[pl.ds(c * _N + s * _RPS, _RPS)])

    @pl.when(s == _NS - 1)
    def _():
        pltpu.sync_copy(acc_sh.at[pl.ds(_NS * _RPS, _TAIL)],
                        out_hbm.at[pl.ds(c * _N + _NS * _RPS, _TAIL)])


_segsum = pl.kernel(
    _segsum_body,
    out_type=jax.ShapeDtypeStruct((_NC * _N, _D), jnp.float32),
    mesh=plsc.VectorSubcoreMesh(core_axis_name="c", subcore_axis_name="s",
                                num_cores=_NC, num_subcores=_NS),
    scratch_types=[
        pltpu.VMEM_SHARED((_N, _D), jnp.float32),
        pltpu.VMEM((_K,), jnp.int32),
        pltpu.VMEM((_K,), jnp.int32),
        pltpu.VMEM((_K, _D), jnp.float32),
        pltpu.SemaphoreType.DMA,
    ],
)


def _mlp_body(hp, p0, p1, w1, b1, w2, b2, h2o, sums):
    t = pl.program_id(0)
    inp = hp[...] + p0[...] + p1[...]
    z = jnp.dot(inp, w1[...], preferred_element_type=jnp.float32) + b1[...]
    z = jnp.maximum(z, 0.0)
    h2 = jnp.dot(z, w2[...], preferred_element_type=jnp.float32) + b2[...]
    h2 = jnp.maximum(h2, 0.0)
    h2o[...] = h2

    @pl.when(t == 0)
    def _():
        sums[...] = jnp.zeros_like(sums)

    row = jnp.concatenate(
        [jnp.sum(h2, axis=0, keepdims=True),
         jnp.sum(h2 * h2, axis=0, keepdims=True),
         jnp.zeros((6, _D), jnp.float32)], axis=0)
    sums[...] += row


_mlp = pl.pallas_call(
    _mlp_body,
    grid=(_NRB,),
    in_specs=[
        pl.BlockSpec((_RB, _D), lambda t: (t, 0)),
        pl.BlockSpec((_RB, _D), lambda t: (t, 0)),
        pl.BlockSpec((_RB, _D), lambda t: (t, 0)),
        pl.BlockSpec((_D, _D), lambda t: (0, 0)),
        pl.BlockSpec((1, _D), lambda t: (0, 0)),
        pl.BlockSpec((_D, _D), lambda t: (0, 0)),
        pl.BlockSpec((1, _D), lambda t: (0, 0)),
    ],
    out_specs=[
        pl.BlockSpec((_RB, _D), lambda t: (t, 0)),
        pl.BlockSpec((8, _D), lambda t: (0, 0)),
    ],
    out_shape=[
        jax.ShapeDtypeStruct((_N, _D), jnp.float32),
        jax.ShapeDtypeStruct((8, _D), jnp.float32),
    ],
)


def _bn_body(h2, sums, gg, bb, out):
    mu = sums[0:1, :] * (1.0 / _N)
    ex2 = sums[1:2, :] * (1.0 / _N)
    var = ex2 - mu * mu
    inv = lax.rsqrt(var + 1e-5)
    out[...] = (h2[...] - mu) * (inv * gg[...]) + bb[...]


_bn = pl.pallas_call(
    _bn_body,
    grid=(_NRB,),
    in_specs=[
        pl.BlockSpec((_RB, _D), lambda t: (t, 0)),
        pl.BlockSpec((8, _D), lambda t: (0, 0)),
        pl.BlockSpec((1, _D), lambda t: (0, 0)),
        pl.BlockSpec((1, _D), lambda t: (0, 0)),
    ],
    out_specs=pl.BlockSpec((_RB, _D), lambda t: (t, 0)),
    out_shape=jax.ShapeDtypeStruct((_N, _D), jnp.float32),
)


def _head_body(b_ref, h0, h1, h2, p1w, pb1, p2w, pb2, yout, acc):
    t = pl.program_id(0)

    @pl.when(t == 0)
    def _():
        acc[...] = jnp.zeros_like(acc)

    bvec = b_ref[0, 0, :]
    iota = lax.broadcasted_iota(jnp.int32, (_G, _RB), 0)
    onehot = (bvec[None, :] == iota).astype(jnp.float32)
    for k, h in enumerate((h0, h1, h2)):
        acc[:, k * _D:(k + 1) * _D] += jnp.dot(
            onehot, h[...], preferred_element_type=jnp.float32)

    @pl.when(t == _NRB - 1)
    def _():
        pool = acc[...]
        z = jnp.dot(pool, p1w[...], preferred_element_type=jnp.float32) + pb1[...]
        z = jnp.maximum(z, 0.0)
        yout[...] = jnp.dot(z, p2w[...], preferred_element_type=jnp.float32) + pb2[...]


_head = pl.pallas_call(
    _head_body,
    grid=(_NRB,),
    in_specs=[
        pl.BlockSpec((1, 1, _RB), lambda t: (t, 0, 0)),
        pl.BlockSpec((_RB, _D), lambda t: (t, 0)),
        pl.BlockSpec((_RB, _D), lambda t: (t, 0)),
        pl.BlockSpec((_RB, _D), lambda t: (t, 0)),
        pl.BlockSpec((3 * _D, 3 * _D), lambda t: (0, 0)),
        pl.BlockSpec((1, 3 * _D), lambda t: (0, 0)),
        pl.BlockSpec((3 * _D, 3 * _D), lambda t: (0, 0)),
        pl.BlockSpec((1, 3 * _D), lambda t: (0, 0)),
    ],
    out_specs=pl.BlockSpec((_G, 3 * _D), lambda t: (0, 0)),
    out_shape=jax.ShapeDtypeStruct((_G, 3 * _D), jnp.float32),
    scratch_shapes=[pltpu.VMEM((_G, 3 * _D), jnp.float32)],
)


def kernel(x, edge_index, batch, num_graphs,
           W1_0, b1_0, W2_0, b2_0, g_0, be_0,
           W1_1, b1_1, W2_1, b2_1, g_1, be_1,
           W1_2, b1_2, W2_2, b2_2, g_2, be_2,
           P1, pb1, P2, pb2):
    src = edge_index[0].astype(jnp.int32)
    dst = edge_index[1].astype(jnp.int32)
    batch3 = batch.astype(jnp.int32).reshape(_NRB, 1, _RB)
    zeros = jnp.zeros((_N, _D), jnp.float32)

    layers = [
        (W1_0, b1_0.reshape(1, _D), W2_0, b2_0.reshape(1, _D),
         g_0.reshape(1, _D), be_0.reshape(1, _D)),
        (W1_1, b1_1.reshape(1, _D), W2_1, b2_1.reshape(1, _D),
         g_1.reshape(1, _D), be_1.reshape(1, _D)),
        (W1_2, b1_2.reshape(1, _D), W2_2, b2_2.reshape(1, _D),
         g_2.reshape(1, _D), be_2.reshape(1, _D)),
    ]

    h = x
    hs = []
    for (w1, b1, w2, b2, g, be) in layers:
        parts = _segsum(h, src, dst, zeros)
        h2, sums = _mlp(h, parts[:_N], parts[_N:], w1, b1, w2, b2)
        h = _bn(h2, sums, g, be)
        hs.append(h)

    y = _head(batch3, hs[0], hs[1], hs[2],
              P1, pb1.reshape(1, 3 * _D), P2, pb2.reshape(1, 3 * _D))
    return y


# speedup vs baseline: 7.1764x; 1.8041x over previous
import functools

import jax
import jax.numpy as jnp
from jax import lax
from jax.experimental import pallas as pl
from jax.experimental.pallas import tpu as pltpu
from jax.experimental.pallas import tpu_sc as plsc

_N = 10000
_E = 320000
_D = 128
_G = 128
_NC = 2
_NS = 16
_NW = _NC * _NS
_K = 80
_NCHUNK = 125
_EPW = _NCHUNK * _K
_NBUF = 3
_NGRP = 41
_RPS = 624
_TAIL = _N - _NS * _RPS

_RB = 400
_NRB = _N // _RB


def _segsum_body(h_hbm, src_hbm, dst_hbm, zeros_hbm, out_hbm,
                 acc_sh, si0, si1, si2, di0, di1, di2, r0, r1, r2,
                 g0, g1, g2, s0, s1, s2, i0, i1, i2):
    sidx = (si0, si1, si2)
    didx = (di0, di1, di2)
    rows = (r0, r1, r2)
    gsem = (g0, g1, g2)
    ssem = (s0, s1, s2)
    isem = (i0, i1, i2)
    c = lax.axis_index("c")
    s = lax.axis_index("s")
    w = s * _NC + c
    base = w * _EPW

    pltpu.sync_copy(zeros_hbm.at[pl.ds(s * _RPS, _RPS)],
                    acc_sh.at[pl.ds(s * _RPS, _RPS)])

    @pl.when(s == _NS - 1)
    def _():
        pltpu.sync_copy(zeros_hbm.at[pl.ds(_NS * _RPS, _TAIL)],
                        acc_sh.at[pl.ds(_NS * _RPS, _TAIL)])

    plsc.subcore_barrier()

    def idxload(j, b):
        off = pl.multiple_of(base + j * _K, 8)
        pltpu.async_copy(src_hbm.at[pl.ds(off, _K)], sidx[b], isem[b])
        pltpu.async_copy(dst_hbm.at[pl.ds(off, _K)], didx[b], isem[b])

    def wait_idx(j, b):
        off = pl.multiple_of(base + j * _K, 8)
        pltpu.make_async_copy(src_hbm.at[pl.ds(off, _K)], sidx[b],
                              isem[b]).wait()
        pltpu.make_async_copy(dst_hbm.at[pl.ds(off, _K)], didx[b],
                              isem[b]).wait()

    def gather(b):
        pltpu.async_copy(h_hbm.at[sidx[b]], rows[b], gsem[b])

    def wait_gather(b):
        pltpu.make_async_copy(h_hbm.at[sidx[b]], rows[b], gsem[b]).wait()

    def scatter(b):
        pltpu.async_copy(rows[b], acc_sh.at[didx[b]], ssem[b], add=True)

    def wait_scatter(b):
        pltpu.make_async_copy(rows[b], acc_sh.at[didx[b]], ssem[b]).wait()

    for b in range(_NBUF):
        idxload(b, b)
    for b in range(_NBUF):
        wait_idx(b, b)
        gather(b)

    def group(gi, carry):
        j0 = gi * _NBUF
        for b in range(_NBUF):
            wait_gather(b)
            scatter(b)
        for b in range(_NBUF):
            wait_scatter(b)

            @pl.when(j0 + _NBUF + b < _NCHUNK)
            def _():
                idxload(j0 + _NBUF + b, b)

        for b in range(_NBUF):
            @pl.when(j0 + _NBUF + b < _NCHUNK)
            def _():
                wait_idx(j0 + _NBUF + b, b)
                gather(b)

        return carry

    lax.fori_loop(0, _NGRP, group, 0)
    for j in range(_NGRP * _NBUF, _NCHUNK):
        b = j % _NBUF
        wait_gather(b)
        scatter(b)
        wait_scatter(b)

    plsc.subcore_barrier()
    pltpu.sync_copy(acc_sh.at[pl.ds(s * _RPS, _RPS)],
                    out_hbm.at[pl.ds(c * _N + s * _RPS, _RPS)])

    @pl.when(s == _NS - 1)
    def _():
        pltpu.sync_copy(acc_sh.at[pl.ds(_NS * _RPS, _TAIL)],
                        out_hbm.at[pl.ds(c * _N + _NS * _RPS, _TAIL)])


_segsum = pl.kernel(
    _segsum_body,
    out_type=jax.ShapeDtypeStruct((_NC * _N, _D), jnp.float32),
    mesh=plsc.VectorSubcoreMesh(core_axis_name="c", subcore_axis_name="s",
                                num_cores=_NC, num_subcores=_NS),
    scratch_types=(
        [pltpu.VMEM_SHARED((_N, _D), jnp.float32)]
        + [pltpu.VMEM((_K,), jnp.int32) for _ in range(2 * _NBUF)]
        + [pltpu.VMEM((_K, _D), jnp.float32) for _ in range(_NBUF)]
        + [pltpu.SemaphoreType.DMA for _ in range(3 * _NBUF)]
    ),
)


def _mlp_body(hp, p0, p1, w1, b1, w2, b2, h2o, sums):
    t = pl.program_id(0)
    inp = hp[...] + p0[...] + p1[...]
    z = jnp.dot(inp, w1[...], preferred_element_type=jnp.float32) + b1[...]
    z = jnp.maximum(z, 0.0)
    h2 = jnp.dot(z, w2[...], preferred_element_type=jnp.float32) + b2[...]
    h2 = jnp.maximum(h2, 0.0)
    h2o[...] = h2

    @pl.when(t == 0)
    def _():
        sums[...] = jnp.zeros_like(sums)

    row = jnp.concatenate(
        [jnp.sum(h2, axis=0, keepdims=True),
         jnp.sum(h2 * h2, axis=0, keepdims=True),
         jnp.zeros((6, _D), jnp.float32)], axis=0)
    sums[...] += row


_mlp = pl.pallas_call(
    _mlp_body,
    grid=(_NRB,),
    in_specs=[
        pl.BlockSpec((_RB, _D), lambda t: (t, 0)),
        pl.BlockSpec((_RB, _D), lambda t: (t, 0)),
        pl.BlockSpec((_RB, _D), lambda t: (t, 0)),
        pl.BlockSpec((_D, _D), lambda t: (0, 0)),
        pl.BlockSpec((1, _D), lambda t: (0, 0)),
        pl.BlockSpec((_D, _D), lambda t: (0, 0)),
        pl.BlockSpec((1, _D), lambda t: (0, 0)),
    ],
    out_specs=[
        pl.BlockSpec((_RB, _D), lambda t: (t, 0)),
        pl.BlockSpec((8, _D), lambda t: (0, 0)),
    ],
    out_shape=[
        jax.ShapeDtypeStruct((_N, _D), jnp.float32),
        jax.ShapeDtypeStruct((8, _D), jnp.float32),
    ],
)


def _bn_body(h2, sums, gg, bb, out):
    mu = sums[0:1, :] * (1.0 / _N)
    ex2 = sums[1:2, :] * (1.0 / _N)
    var = ex2 - mu * mu
    inv = lax.rsqrt(var + 1e-5)
    out[...] = (h2[...] - mu) * (inv * gg[...]) + bb[...]


_bn = pl.pallas_call(
    _bn_body,
    grid=(_NRB,),
    in_specs=[
        pl.BlockSpec((_RB, _D), lambda t: (t, 0)),
        pl.BlockSpec((8, _D), lambda t: (0, 0)),
        pl.BlockSpec((1, _D), lambda t: (0, 0)),
        pl.BlockSpec((1, _D), lambda t: (0, 0)),
    ],
    out_specs=pl.BlockSpec((_RB, _D), lambda t: (t, 0)),
    out_shape=jax.ShapeDtypeStruct((_N, _D), jnp.float32),
)


def _head_body(b_ref, h0, h1, h2, p1w, pb1, p2w, pb2, yout, acc):
    t = pl.program_id(0)

    @pl.when(t == 0)
    def _():
        acc[...] = jnp.zeros_like(acc)

    bvec = b_ref[0, 0, :]
    iota = lax.broadcasted_iota(jnp.int32, (_G, _RB), 0)
    onehot = (bvec[None, :] == iota).astype(jnp.float32)
    for k, h in enumerate((h0, h1, h2)):
        acc[:, k * _D:(k + 1) * _D] += jnp.dot(
            onehot, h[...], preferred_element_type=jnp.float32)

    @pl.when(t == _NRB - 1)
    def _():
        pool = acc[...]
        z = jnp.dot(pool, p1w[...], preferred_element_type=jnp.float32) + pb1[...]
        z = jnp.maximum(z, 0.0)
        yout[...] = jnp.dot(z, p2w[...], preferred_element_type=jnp.float32) + pb2[...]


_head = pl.pallas_call(
    _head_body,
    grid=(_NRB,),
    in_specs=[
        pl.BlockSpec((1, 1, _RB), lambda t: (t, 0, 0)),
        pl.BlockSpec((_RB, _D), lambda t: (t, 0)),
        pl.BlockSpec((_RB, _D), lambda t: (t, 0)),
        pl.BlockSpec((_RB, _D), lambda t: (t, 0)),
        pl.BlockSpec((3 * _D, 3 * _D), lambda t: (0, 0)),
        pl.BlockSpec((1, 3 * _D), lambda t: (0, 0)),
        pl.BlockSpec((3 * _D, 3 * _D), lambda t: (0, 0)),
        pl.BlockSpec((1, 3 * _D), lambda t: (0, 0)),
    ],
    out_specs=pl.BlockSpec((_G, 3 * _D), lambda t: (0, 0)),
    out_shape=jax.ShapeDtypeStruct((_G, 3 * _D), jnp.float32),
    scratch_shapes=[pltpu.VMEM((_G, 3 * _D), jnp.float32)],
)


def kernel(x, edge_index, batch, num_graphs,
           W1_0, b1_0, W2_0, b2_0, g_0, be_0,
           W1_1, b1_1, W2_1, b2_1, g_1, be_1,
           W1_2, b1_2, W2_2, b2_2, g_2, be_2,
           P1, pb1, P2, pb2):
    src = edge_index[0].astype(jnp.int32)
    dst = edge_index[1].astype(jnp.int32)
    batch3 = batch.astype(jnp.int32).reshape(_NRB, 1, _RB)
    zeros = jnp.zeros((_N, _D), jnp.float32)

    layers = [
        (W1_0, b1_0.reshape(1, _D), W2_0, b2_0.reshape(1, _D),
         g_0.reshape(1, _D), be_0.reshape(1, _D)),
        (W1_1, b1_1.reshape(1, _D), W2_1, b2_1.reshape(1, _D),
         g_1.reshape(1, _D), be_1.reshape(1, _D)),
        (W1_2, b1_2.reshape(1, _D), W2_2, b2_2.reshape(1, _D),
         g_2.reshape(1, _D), be_2.reshape(1, _D)),
    ]

    h = x
    hs = []
    for (w1, b1, w2, b2, g, be) in layers:
        parts = _segsum(h, src, dst, zeros)
        h2, sums = _mlp(h, parts[:_N], parts[_N:], w1, b1, w2, b2)
        h = _bn(h2, sums, g, be)
        hs.append(h)

    y = _head(batch3, hs[0], hs[1], hs[2],
              P1, pb1.reshape(1, 3 * _D), P2, pb2.reshape(1, 3 * _D))
    return y
